# baseline (speedup 1.0000x reference)
import jax
import jax.numpy as jnp
from jax import lax
from jax.experimental import pallas as pl
from jax.experimental.pallas import tpu as pltpu
from jax.experimental.pallas import tpu_sc as plsc

_ROWS = 16384
_COLS = 100
_N = _ROWS * _COLS
_V = 1000000
_NW = 32
_PER_W = _N // _NW
_CHUNK = 6400
_NCHUNK = _PER_W // _CHUNK
_W_BITS = 15
_WORDS = 1 << _W_BITS
_NBITS = 31
_PAD_V = _NBITS * _WORDS + _WORDS
_WPT = _WORDS // 16


_WPW = _WORDS // _NW


def _pack_body(table_hbm, packed_hbm, colbuf, chunk, psem):
    c = lax.axis_index("c")
    s = lax.axis_index("s")
    wid = c * jnp.int32(16) + s
    wbase = wid * jnp.int32(_WPW)

    copies = [
        pltpu.async_copy(
            table_hbm.at[pl.ds(wbase + jnp.int32(b * _WORDS), _WPW)],
            colbuf.at[pl.ds(b * _WPW, _WPW)], psem)
        for b in range(_NBITS)
    ]
    for cp in copies:
        cp.wait()

    def pack_loop(g, o):
        acc = jnp.zeros((16,), jnp.int32)
        for b in range(_NBITS):
            v = colbuf[pl.ds(jnp.int32(b * _WPW) + o, 16)]
            acc = acc | jnp.where(v != 0.0,
                                  jnp.int32(1 << b), jnp.int32(0))
        chunk[pl.ds(o, 16)] = acc
        return o + jnp.int32(16)

    lax.fori_loop(0, _WPW // 16, pack_loop, jnp.int32(0))
    pltpu.sync_copy(chunk, packed_hbm.at[pl.ds(wbase, _WPW)])


def _lookup_sc_body(packed_hbm, idx2_hbm, out2_hbm,
                    packed, idxbufs, outbufs, psem, isems, osems):
    c = lax.axis_index("c")
    s = lax.axis_index("s")
    wid = c * jnp.int32(16) + s
    base = wid * jnp.int32(_PER_W)

    ic = {0: pltpu.async_copy(
        idx2_hbm.at[pl.ds(base, _CHUNK)], idxbufs[0], isems[0])}
    pltpu.async_copy(packed_hbm, packed, psem).wait()

    def make_lookup(idxbuf, outbuf):
        def lookup_body(i, o):
            for u in range(4):
                oo = o + jnp.int32(16 * u)
                ivec = plsc.bitcast(idxbuf[pl.ds(oo, 16)], jnp.int32)
                w = ivec & jnp.int32(_WORDS - 1)
                b = lax.shift_right_logical(ivec, jnp.int32(_W_BITS))
                word = plsc.load_gather(packed, [w])
                bit = lax.shift_right_logical(word, b) & jnp.int32(1)
                outbuf[pl.ds(oo, 16)] = plsc.bitcast(
                    bit ^ jnp.int32(1), jnp.uint32)
            return o + jnp.int32(64)
        return lookup_body

    oc = {}
    for cc in range(_NCHUNK):
        nb = cc & 1
        if cc + 1 < _NCHUNK:
            ic[cc + 1] = pltpu.async_copy(
                idx2_hbm.at[pl.ds(base + jnp.int32((cc + 1) * _CHUNK),
                                  _CHUNK)],
                idxbufs[(cc + 1) & 1], isems[(cc + 1) & 1])
        ic[cc].wait()
        if cc >= 2:
            oc[cc - 2].wait()
        lax.fori_loop(0, _CHUNK // 64,
                      make_lookup(idxbufs[nb], outbufs[nb]), jnp.int32(0))
        oc[cc] = pltpu.async_copy(
            outbufs[nb],
            out2_hbm.at[pl.ds(base + jnp.int32(cc * _CHUNK), _CHUNK)],
            osems[nb])
    oc[_NCHUNK - 2].wait()
    oc[_NCHUNK - 1].wait()


def kernel(donors_idx, mask_fit_X_col):
    idx = donors_idx.T.astype(jnp.uint32).reshape(_N)
    table = jnp.concatenate(
        [mask_fit_X_col.astype(jnp.float32),
         jnp.zeros((_PAD_V - _V,), jnp.float32)])

    mesh = plsc.VectorSubcoreMesh(core_axis_name="c", subcore_axis_name="s")
    packed = pl.kernel(
        _pack_body,
        out_type=jax.ShapeDtypeStruct((_WORDS,), jnp.int32),
        mesh=mesh,
        compiler_params=pltpu.CompilerParams(needs_layout_passes=False),
        scratch_types=[
            pltpu.VMEM((_NBITS * _WPW,), jnp.float32),
            pltpu.VMEM((_WPW,), jnp.int32),
            pltpu.SemaphoreType.DMA,
        ],
    )(table)
    out = pl.kernel(
        _lookup_sc_body,
        out_type=jax.ShapeDtypeStruct((_N,), jnp.uint32),
        mesh=mesh,
        compiler_params=pltpu.CompilerParams(needs_layout_passes=False),
        scratch_types=[
            pltpu.VMEM((_WORDS,), jnp.int32),
            [pltpu.VMEM((_CHUNK,), jnp.uint32)] * 2,
            [pltpu.VMEM((_CHUNK,), jnp.uint32)] * 2,
            pltpu.SemaphoreType.DMA,
            [pltpu.SemaphoreType.DMA] * 2,
            [pltpu.SemaphoreType.DMA] * 2,
        ],
    )(packed, idx)
    return out.reshape(_COLS, _ROWS).astype(donors_idx.dtype).T


# speedup vs baseline: 1.0146x; 1.0146x over previous
import jax
import jax.numpy as jnp
from jax import lax
from jax.experimental import pallas as pl
from jax.experimental.pallas import tpu as pltpu
from jax.experimental.pallas import tpu_sc as plsc

_ROWS = 16384
_COLS = 100
_N = _ROWS * _COLS
_V = 1000000
_NW = 32
_PER_W = _N // _NW
_CHUNK = 6400
_NCHUNK = _PER_W // _CHUNK
_W_BITS = 15
_WORDS = 1 << _W_BITS
_NBITS = 31
_PAD_V = _NBITS * _WORDS + _WORDS
_WPT = _WORDS // 16


_WPW = _WORDS // _NW


def _pack_body(table_hbm, packed_hbm, colbuf, chunk, psem):
    c = lax.axis_index("c")
    s = lax.axis_index("s")
    wid = c * jnp.int32(16) + s
    wbase = wid * jnp.int32(_WPW)

    copies = [
        pltpu.async_copy(
            table_hbm.at[pl.ds(wbase + jnp.int32(b * _WORDS), _WPW)],
            colbuf.at[pl.ds(b * _WPW, _WPW)], psem)
        for b in range(_NBITS - 1)
    ]

    zeros16f = jnp.zeros((16,), jnp.float32)
    tail_lo = 30 * _WORDS
    full_w = (_V - tail_lo) // _WPW
    part_n = _V - tail_lo - full_w * _WPW

    def zero_tail(start, count):
        def zbody(g, o):
            colbuf[pl.ds(o, 16)] = zeros16f
            return o + jnp.int32(16)
        lax.fori_loop(0, count // 16, zbody, jnp.int32(30 * _WPW + start))

    @pl.when(wid < full_w)
    def _():
        pltpu.sync_copy(
            table_hbm.at[pl.ds(wbase + jnp.int32(tail_lo), _WPW)],
            colbuf.at[pl.ds(30 * _WPW, _WPW)])

    @pl.when(wid == full_w)
    def _():
        pltpu.sync_copy(
            table_hbm.at[pl.ds(wbase + jnp.int32(tail_lo), part_n)],
            colbuf.at[pl.ds(30 * _WPW, part_n)])
        zero_tail(part_n, _WPW - part_n)

    @pl.when(wid > full_w)
    def _():
        zero_tail(0, _WPW)

    for cp in copies:
        cp.wait()

    def pack_loop(g, o):
        acc = jnp.zeros((16,), jnp.int32)
        for b in range(_NBITS):
            v = colbuf[pl.ds(jnp.int32(b * _WPW) + o, 16)]
            acc = acc | jnp.where(v != 0.0,
                                  jnp.int32(1 << b), jnp.int32(0))
        chunk[pl.ds(o, 16)] = acc
        return o + jnp.int32(16)

    lax.fori_loop(0, _WPW // 16, pack_loop, jnp.int32(0))
    pltpu.sync_copy(chunk, packed_hbm.at[pl.ds(wbase, _WPW)])


def _lookup_sc_body(packed_hbm, idx2_hbm, out2_hbm,
                    packed, idxbufs, outbufs, psem, isems, osems):
    c = lax.axis_index("c")
    s = lax.axis_index("s")
    wid = c * jnp.int32(16) + s
    base = wid * jnp.int32(_PER_W)

    ic = {0: pltpu.async_copy(
        idx2_hbm.at[pl.ds(base, _CHUNK)], idxbufs[0], isems[0])}
    pltpu.async_copy(packed_hbm, packed, psem).wait()

    def make_lookup(idxbuf, outbuf):
        def lookup_body(i, o):
            for u in range(4):
                oo = o + jnp.int32(16 * u)
                ivec = plsc.bitcast(idxbuf[pl.ds(oo, 16)], jnp.int32)
                w = ivec & jnp.int32(_WORDS - 1)
                b = lax.shift_right_logical(ivec, jnp.int32(_W_BITS))
                word = plsc.load_gather(packed, [w])
                bit = lax.shift_right_logical(word, b) & jnp.int32(1)
                outbuf[pl.ds(oo, 16)] = plsc.bitcast(
                    bit ^ jnp.int32(1), jnp.uint32)
            return o + jnp.int32(64)
        return lookup_body

    oc = {}
    for cc in range(_NCHUNK):
        nb = cc & 1
        if cc + 1 < _NCHUNK:
            ic[cc + 1] = pltpu.async_copy(
                idx2_hbm.at[pl.ds(base + jnp.int32((cc + 1) * _CHUNK),
                                  _CHUNK)],
                idxbufs[(cc + 1) & 1], isems[(cc + 1) & 1])
        ic[cc].wait()
        if cc >= 2:
            oc[cc - 2].wait()
        lax.fori_loop(0, _CHUNK // 64,
                      make_lookup(idxbufs[nb], outbufs[nb]), jnp.int32(0))
        oc[cc] = pltpu.async_copy(
            outbufs[nb],
            out2_hbm.at[pl.ds(base + jnp.int32(cc * _CHUNK), _CHUNK)],
            osems[nb])
    oc[_NCHUNK - 2].wait()
    oc[_NCHUNK - 1].wait()


def kernel(donors_idx, mask_fit_X_col):
    idx = donors_idx.T.astype(jnp.uint32).reshape(_N)
    table = mask_fit_X_col.astype(jnp.float32)

    mesh = plsc.VectorSubcoreMesh(core_axis_name="c", subcore_axis_name="s")
    packed = pl.kernel(
        _pack_body,
        out_type=jax.ShapeDtypeStruct((_WORDS,), jnp.int32),
        mesh=mesh,
        compiler_params=pltpu.CompilerParams(needs_layout_passes=False),
        scratch_types=[
            pltpu.VMEM((_NBITS * _WPW,), jnp.float32),
            pltpu.VMEM((_WPW,), jnp.int32),
            pltpu.SemaphoreType.DMA,
        ],
    )(table)
    out = pl.kernel(
        _lookup_sc_body,
        out_type=jax.ShapeDtypeStruct((_N,), jnp.uint32),
        mesh=mesh,
        compiler_params=pltpu.CompilerParams(needs_layout_passes=False),
        scratch_types=[
            pltpu.VMEM((_WORDS,), jnp.int32),
            [pltpu.VMEM((_CHUNK,), jnp.uint32)] * 2,
            [pltpu.VMEM((_CHUNK,), jnp.uint32)] * 2,
            pltpu.SemaphoreType.DMA,
            [pltpu.SemaphoreType.DMA] * 2,
            [pltpu.SemaphoreType.DMA] * 2,
        ],
    )(packed, idx)
    return out.reshape(_COLS, _ROWS).astype(donors_idx.dtype).T
